# baseline (device time: 162702 ns/iter reference)
import jax
import jax.numpy as jnp
from jax import lax
from jax.experimental import pallas as pl
from jax.experimental.pallas import tpu as pltpu

N_DEV = 4
C = 2


def kernel(x, w_mat):
    m_per, k = x.shape
    _, n_per = w_mat.shape
    half = m_per // 2
    sub = half // C
    n_hops = N_DEV - 1
    n_slots = n_hops * C

    def body(x_ref, w_hbm, out_ref, own_ref, wbf_ref, cw_ref, ccw_ref,
             amax_ref, w_sem, cw_send, cw_recv, ccw_send, ccw_recv,
             a_send, a_recv):
        my = lax.axis_index("i")
        left = (my + N_DEV - 1) % N_DEV
        right = (my + 1) % N_DEV

        barrier_sem = pltpu.get_barrier_semaphore()
        for nbr in (left, right):
            pl.semaphore_signal(
                barrier_sem, inc=1,
                device_id=(nbr,), device_id_type=pl.DeviceIdType.MESH,
            )
        pl.semaphore_wait(barrier_sem, 2)

        w_copy = pltpu.make_async_copy(w_hbm, out_ref, w_sem)
        w_copy.start()

        def gemm_into(chunk, row0, rows):
            y = jnp.dot(chunk, wbf_ref[...],
                        preferred_element_type=jnp.float32)
            y = jnp.maximum(y, 0.0)
            out_ref[pl.ds(row0, rows), :] = y
            return jnp.max(y)

        def slot(h, c):
            return h * C + c

        def mk(src, dst, send_sems, recv_sems, h, c, dev):
            return pltpu.make_async_remote_copy(
                src_ref=src,
                dst_ref=dst.at[slot(h, c)],
                send_sem=send_sems.at[slot(h, c)],
                recv_sem=recv_sems.at[slot(h, c)],
                device_id=(dev,),
                device_id_type=pl.DeviceIdType.MESH,
            )

        cw_rdmas = {}
        ccw_rdmas = {}
        for c in range(C):
            top = pl.ds(c * sub, sub)
            own_ref[top, :] = x_ref[top, :].astype(jnp.bfloat16)
            r = mk(own_ref.at[top], cw_ref, cw_send, cw_recv, 0, c, right)
            r.start()
            cw_rdmas[(0, c)] = r
            bot = pl.ds(half + c * sub, sub)
            own_ref[bot, :] = x_ref[bot, :].astype(jnp.bfloat16)
            r = mk(own_ref.at[bot], ccw_ref, ccw_send, ccw_recv, 0, c, left)
            r.start()
            ccw_rdmas[(0, c)] = r

        w_copy.wait()
        wbf_ref[...] = out_ref[...].astype(jnp.bfloat16)

        amax = gemm_into(own_ref[...], my * m_per, m_per)

        for h in range(1, n_hops):
            for c in range(C):
                cw_rdmas[(h - 1, c)].wait_recv()
                r = mk(cw_ref.at[slot(h - 1, c)], cw_ref,
                       cw_send, cw_recv, h, c, right)
                r.start()
                cw_rdmas[(h, c)] = r
                ccw_rdmas[(h - 1, c)].wait_recv()
                r = mk(ccw_ref.at[slot(h - 1, c)], ccw_ref,
                       ccw_send, ccw_recv, h, c, left)
                r.start()
                ccw_rdmas[(h, c)] = r
            cw_org = (my + N_DEV - h) % N_DEV
            ccw_org = (my + h) % N_DEV
            for c in range(C):
                amax = jnp.maximum(amax, gemm_into(
                    cw_ref[slot(h - 1, c)], cw_org * m_per + c * sub, sub))
                amax = jnp.maximum(amax, gemm_into(
                    ccw_ref[slot(h - 1, c)],
                    ccw_org * m_per + half + c * sub, sub))

        cw_org = (my + 1) % N_DEV
        ccw_org = left
        for c in range(C):
            cw_rdmas[(n_hops - 1, c)].wait_recv()
            amax = jnp.maximum(amax, gemm_into(
                cw_ref[slot(n_hops - 1, c)], cw_org * m_per + c * sub, sub))
            ccw_rdmas[(n_hops - 1, c)].wait_recv()
            amax = jnp.maximum(amax, gemm_into(
                ccw_ref[slot(n_hops - 1, c)],
                ccw_org * m_per + half + c * sub, sub))

        amax_ref[N_DEV - 1] = jnp.full((8, 128), amax, jnp.float32)
        a_rdmas = []
        for kk in range(1, N_DEV):
            r = pltpu.make_async_remote_copy(
                src_ref=amax_ref.at[N_DEV - 1],
                dst_ref=amax_ref.at[kk - 1],
                send_sem=a_send.at[kk - 1],
                recv_sem=a_recv.at[kk - 1],
                device_id=((my + kk) % N_DEV,),
                device_id_type=pl.DeviceIdType.MESH,
            )
            r.start()
            a_rdmas.append(r)

        for r in cw_rdmas.values():
            r.wait_send()
        for r in ccw_rdmas.values():
            r.wait_send()

        for r in a_rdmas:
            r.wait()
        gmax = amax
        for s in range(N_DEV - 1):
            gmax = jnp.maximum(gmax, amax_ref[s, 0, 0])

        scale = gmax / 448.0
        inv = 448.0 / gmax
        q = jnp.minimum(out_ref[...] * inv, 448.0).astype(jnp.float8_e4m3fn)
        out_ref[...] = q.astype(jnp.float32) * scale

    return pl.pallas_call(
        body,
        out_shape=jax.ShapeDtypeStruct((N_DEV * m_per, n_per), jnp.float32),
        in_specs=[
            pl.BlockSpec(memory_space=pltpu.VMEM),
            pl.BlockSpec(memory_space=pltpu.MemorySpace.HBM),
        ],
        out_specs=pl.BlockSpec(memory_space=pltpu.VMEM),
        scratch_shapes=[
            pltpu.VMEM((m_per, k), jnp.bfloat16),
            pltpu.VMEM((k, n_per), jnp.bfloat16),
            pltpu.VMEM((n_slots, sub, k), jnp.bfloat16),
            pltpu.VMEM((n_slots, sub, k), jnp.bfloat16),
            pltpu.VMEM((N_DEV, 8, 128), jnp.float32),
            pltpu.SemaphoreType.DMA,
            pltpu.SemaphoreType.DMA((n_slots,)),
            pltpu.SemaphoreType.DMA((n_slots,)),
            pltpu.SemaphoreType.DMA((n_slots,)),
            pltpu.SemaphoreType.DMA((n_slots,)),
            pltpu.SemaphoreType.DMA((N_DEV - 1,)),
            pltpu.SemaphoreType.DMA((N_DEV - 1,)),
        ],
        compiler_params=pltpu.CompilerParams(
            collective_id=0, vmem_limit_bytes=100 * 1024 * 1024),
    )(x, w_mat)


# device time: 161510 ns/iter; 1.0074x vs baseline; 1.0074x over previous
import jax
import jax.numpy as jnp
from jax import lax
from jax.experimental import pallas as pl
from jax.experimental.pallas import tpu as pltpu

N_DEV = 4
C = 2


def kernel(x, w_mat):
    m_per, k = x.shape
    _, n_per = w_mat.shape
    half = m_per // 2
    sub = half // C
    n_hops = N_DEV - 1
    n_slots = n_hops * C

    def body(x_hbm, w_hbm, out_ref, xstage_ref, own_ref, wbf_ref,
             cw_ref, ccw_ref, amax_ref, x_sems, w_sem,
             cw_send, cw_recv, ccw_send, ccw_recv, a_send, a_recv):
        my = lax.axis_index("i")
        left = (my + N_DEV - 1) % N_DEV
        right = (my + 1) % N_DEV

        barrier_sem = pltpu.get_barrier_semaphore()
        for nbr in (left, right):
            pl.semaphore_signal(
                barrier_sem, inc=1,
                device_id=(nbr,), device_id_type=pl.DeviceIdType.MESH,
            )
        pl.semaphore_wait(barrier_sem, 2)

        w_copy = pltpu.make_async_copy(w_hbm, out_ref, w_sem)
        w_copy.start()

        def gemm_into(chunk, row0, rows):
            y = jnp.dot(chunk, wbf_ref[...],
                        preferred_element_type=jnp.float32)
            y = jnp.maximum(y, 0.0)
            out_ref[pl.ds(row0, rows), :] = y
            return jnp.max(y)

        def slot(h, c):
            return h * C + c

        def mk(src, dst, send_sems, recv_sems, h, c, dev):
            return pltpu.make_async_remote_copy(
                src_ref=src,
                dst_ref=dst.at[slot(h, c)],
                send_sem=send_sems.at[slot(h, c)],
                recv_sem=recv_sems.at[slot(h, c)],
                device_id=(dev,),
                device_id_type=pl.DeviceIdType.MESH,
            )

        units = []
        for c in range(C):
            units.append((c * sub, True, c))
            units.append((half + c * sub, False, c))

        for i in range(2):
            pltpu.make_async_copy(
                x_hbm.at[pl.ds(units[i][0], sub)], xstage_ref.at[i],
                x_sems.at[i]).start()

        cw_rdmas = {}
        ccw_rdmas = {}
        for i, (row0, is_cw, c) in enumerate(units):
            buf = i % 2
            pltpu.make_async_copy(
                x_hbm.at[pl.ds(row0, sub)], xstage_ref.at[buf],
                x_sems.at[buf]).wait()
            own_ref[pl.ds(row0, sub), :] = (
                xstage_ref[buf].astype(jnp.bfloat16))
            if i + 2 < len(units):
                pltpu.make_async_copy(
                    x_hbm.at[pl.ds(units[i + 2][0], sub)],
                    xstage_ref.at[buf], x_sems.at[buf]).start()
            if is_cw:
                r = mk(own_ref.at[pl.ds(row0, sub)], cw_ref,
                       cw_send, cw_recv, 0, c, right)
                r.start()
                cw_rdmas[(0, c)] = r
            else:
                r = mk(own_ref.at[pl.ds(row0, sub)], ccw_ref,
                       ccw_send, ccw_recv, 0, c, left)
                r.start()
                ccw_rdmas[(0, c)] = r

        w_copy.wait()
        wbf_ref[...] = out_ref[...].astype(jnp.bfloat16)

        amax = gemm_into(own_ref[...], my * m_per, m_per)

        for h in range(1, n_hops):
            for c in range(C):
                cw_rdmas[(h - 1, c)].wait_recv()
                r = mk(cw_ref.at[slot(h - 1, c)], cw_ref,
                       cw_send, cw_recv, h, c, right)
                r.start()
                cw_rdmas[(h, c)] = r
                ccw_rdmas[(h - 1, c)].wait_recv()
                r = mk(ccw_ref.at[slot(h - 1, c)], ccw_ref,
                       ccw_send, ccw_recv, h, c, left)
                r.start()
                ccw_rdmas[(h, c)] = r
            cw_org = (my + N_DEV - h) % N_DEV
            ccw_org = (my + h) % N_DEV
            for c in range(C):
                amax = jnp.maximum(amax, gemm_into(
                    cw_ref[slot(h - 1, c)], cw_org * m_per + c * sub, sub))
                amax = jnp.maximum(amax, gemm_into(
                    ccw_ref[slot(h - 1, c)],
                    ccw_org * m_per + half + c * sub, sub))

        cw_org = (my + 1) % N_DEV
        ccw_org = left
        for c in range(C):
            cw_rdmas[(n_hops - 1, c)].wait_recv()
            amax = jnp.maximum(amax, gemm_into(
                cw_ref[slot(n_hops - 1, c)], cw_org * m_per + c * sub, sub))
            ccw_rdmas[(n_hops - 1, c)].wait_recv()
            amax = jnp.maximum(amax, gemm_into(
                ccw_ref[slot(n_hops - 1, c)],
                ccw_org * m_per + half + c * sub, sub))

        amax_ref[N_DEV - 1] = jnp.full((8, 128), amax, jnp.float32)
        a_rdmas = []
        for kk in range(1, N_DEV):
            r = pltpu.make_async_remote_copy(
                src_ref=amax_ref.at[N_DEV - 1],
                dst_ref=amax_ref.at[kk - 1],
                send_sem=a_send.at[kk - 1],
                recv_sem=a_recv.at[kk - 1],
                device_id=((my + kk) % N_DEV,),
                device_id_type=pl.DeviceIdType.MESH,
            )
            r.start()
            a_rdmas.append(r)

        for r in cw_rdmas.values():
            r.wait_send()
        for r in ccw_rdmas.values():
            r.wait_send()

        for r in a_rdmas:
            r.wait()
        gmax = amax
        for s in range(N_DEV - 1):
            gmax = jnp.maximum(gmax, amax_ref[s, 0, 0])

        scale = gmax / 448.0
        inv = 448.0 / gmax
        q = jnp.minimum(out_ref[...] * inv, 448.0).astype(jnp.float8_e4m3fn)
        out_ref[...] = q.astype(jnp.float32) * scale

    return pl.pallas_call(
        body,
        out_shape=jax.ShapeDtypeStruct((N_DEV * m_per, n_per), jnp.float32),
        in_specs=[
            pl.BlockSpec(memory_space=pltpu.MemorySpace.HBM),
            pl.BlockSpec(memory_space=pltpu.MemorySpace.HBM),
        ],
        out_specs=pl.BlockSpec(memory_space=pltpu.VMEM),
        scratch_shapes=[
            pltpu.VMEM((2, sub, k), jnp.float32),
            pltpu.VMEM((m_per, k), jnp.bfloat16),
            pltpu.VMEM((k, n_per), jnp.bfloat16),
            pltpu.VMEM((n_slots, sub, k), jnp.bfloat16),
            pltpu.VMEM((n_slots, sub, k), jnp.bfloat16),
            pltpu.VMEM((N_DEV, 8, 128), jnp.float32),
            pltpu.SemaphoreType.DMA((2,)),
            pltpu.SemaphoreType.DMA,
            pltpu.SemaphoreType.DMA((n_slots,)),
            pltpu.SemaphoreType.DMA((n_slots,)),
            pltpu.SemaphoreType.DMA((n_slots,)),
            pltpu.SemaphoreType.DMA((n_slots,)),
            pltpu.SemaphoreType.DMA((N_DEV - 1,)),
            pltpu.SemaphoreType.DMA((N_DEV - 1,)),
        ],
        compiler_params=pltpu.CompilerParams(
            collective_id=0, vmem_limit_bytes=100 * 1024 * 1024),
    )(x, w_mat)
